# baseline (speedup 1.0000x reference)
import functools

import jax
import jax.numpy as jnp
from jax import lax
from jax.experimental import pallas as pl
from jax.experimental.pallas import tpu as pltpu
from jax.experimental.pallas import tpu_sc as plsc

TOP_K = 8
NUM_EXPERTS = 64
BLK = 2048
N_TOKENS = 16384


def _router_body(x_ref, w_ref, b_ref, idx_ref, wgt_ref):
    x = x_ref[...]
    w = w_ref[...]
    logits = jax.lax.dot_general(
        x, w, (((1,), (0,)), ((), ())),
        preferred_element_type=jnp.float32,
    ) + b_ref[...]

    lanes = jax.lax.broadcasted_iota(jnp.int32, logits.shape, 1)
    neg_inf = jnp.float32(-jnp.inf)
    cur = logits
    vals, idxs = [], []
    for _ in range(TOP_K):
        m = jnp.max(cur, axis=1, keepdims=True)
        idx = jnp.argmax(cur, axis=1).reshape(-1, 1)
        vals.append(m)
        idxs.append(idx)
        cur = jnp.where(lanes == idx, neg_inf, cur)

    v = jnp.concatenate(vals, axis=1)
    e = jnp.exp(v - v[:, :1])
    wgt_ref[...] = e / jnp.sum(e, axis=1, keepdims=True)
    idx_ref[...] = jnp.concatenate(idxs, axis=1)


def _fused_tc(x_flat, w_t, bias):
    n, dim = x_flat.shape
    grid = (n // BLK,)
    return pl.pallas_call(
        _router_body,
        grid=grid,
        in_specs=[
            pl.BlockSpec((BLK, dim), lambda i: (i, 0)),
            pl.BlockSpec((dim, NUM_EXPERTS), lambda i: (0, 0)),
            pl.BlockSpec((1, NUM_EXPERTS), lambda i: (0, 0)),
        ],
        out_specs=[
            pl.BlockSpec((BLK, TOP_K), lambda i: (i, 0)),
            pl.BlockSpec((BLK, TOP_K), lambda i: (i, 0)),
        ],
        out_shape=[
            jax.ShapeDtypeStruct((n, TOP_K), jnp.int32),
            jax.ShapeDtypeStruct((n, TOP_K), jnp.float32),
        ],
        compiler_params=pltpu.CompilerParams(
            dimension_semantics=("arbitrary",),
        ),
    )(x_flat, w_t, bias)


def _matmul_body(x_ref, w_ref, b_ref, out_ref):
    out_ref[...] = jax.lax.dot_general(
        x_ref[...], w_ref[...], (((1,), (0,)), ((), ())),
        preferred_element_type=jnp.float32,
    ) + b_ref[...]


def _tc_logits(x_flat, w_t, bias):
    n, dim = x_flat.shape
    return pl.pallas_call(
        _matmul_body,
        grid=(n // BLK,),
        in_specs=[
            pl.BlockSpec((BLK, dim), lambda i: (i, 0)),
            pl.BlockSpec((dim, NUM_EXPERTS), lambda i: (0, 0)),
            pl.BlockSpec((1, NUM_EXPERTS), lambda i: (0, 0)),
        ],
        out_specs=pl.BlockSpec((BLK, NUM_EXPERTS), lambda i: (i, 0)),
        out_shape=jax.ShapeDtypeStruct((n, NUM_EXPERTS), jnp.float32),
        compiler_params=pltpu.CompilerParams(
            dimension_semantics=("arbitrary",),
        ),
    )(x_flat, w_t, bias)


_SCALE = 2097152.0
_NW = 32
_ROWS_W = N_TOKENS // _NW


def _sc_topk_call(logits_flat):
    mesh = plsc.VectorSubcoreMesh(core_axis_name="c", subcore_axis_name="s")
    rows = _ROWS_W

    @functools.partial(
        pl.kernel,
        mesh=mesh,
        out_type=[
            jax.ShapeDtypeStruct((N_TOKENS * 16,), jnp.int32),
            jax.ShapeDtypeStruct((N_TOKENS * 16,), jnp.float32),
        ],
        scratch_types=[
            pltpu.VMEM((rows * 64,), jnp.float32),
            pltpu.VMEM((rows * 16,), jnp.int32),
            pltpu.VMEM((rows * 16,), jnp.float32),
        ],
        compiler_params=pltpu.CompilerParams(needs_layout_passes=False),
    )
    def sc_topk(logits_hbm, idx_hbm, wgt_hbm, log_v, idx_v, wgt_v):
        wid = lax.axis_index("s") * 2 + lax.axis_index("c")
        base = wid * rows
        pltpu.sync_copy(logits_hbm.at[pl.ds(base * 64, rows * 64)], log_v)

        lane = lax.iota(jnp.int32, 16)
        mask8 = lane < 8

        def one_row(r, _):
            chunks = []
            for c in range(4):
                v = log_v[pl.ds(r * 64 + c * 16, 16)]
                v = jnp.minimum(jnp.maximum(v, -15.9), 15.9)
                q = (v * _SCALE).astype(jnp.int32)
                key = q * 64 + (63 - (lane + c * 16))
                skey, _sv = plsc.sort_key_val(key, key, descending=True)
                chunks.append(skey)
            t01 = jnp.maximum(chunks[0], lax.rev(chunks[1], (0,)))
            t23 = jnp.maximum(chunks[2], lax.rev(chunks[3], (0,)))
            t01, _sv = plsc.sort_key_val(t01, t01, descending=True)
            t23, _sv = plsc.sort_key_val(t23, t23, descending=True)
            f = jnp.maximum(t01, lax.rev(t23, (0,)))
            f, _sv = plsc.sort_key_val(f, f, descending=True)

            idx = 63 - (f & 63)
            vv = (f >> 6).astype(jnp.float32) * (1.0 / _SCALE)
            e = jnp.exp(vv - jnp.max(vv))
            e = jnp.where(mask8, e, 0.0)
            w = e / jnp.sum(e)
            idx_v[pl.ds(r * 16, 16)] = idx
            wgt_v[pl.ds(r * 16, 16)] = w
            return _

        lax.fori_loop(0, rows, one_row, 0)
        pltpu.sync_copy(idx_v, idx_hbm.at[pl.ds(base * 16, rows * 16)])
        pltpu.sync_copy(wgt_v, wgt_hbm.at[pl.ds(base * 16, rows * 16)])

    return sc_topk(logits_flat)


def _hybrid_sc(x_flat, w_t, bias):
    logits = _tc_logits(x_flat, w_t, bias)
    idx16, wgt16 = _sc_topk_call(logits.reshape(-1))
    idx = idx16.reshape(N_TOKENS, 16)[:, :TOP_K]
    wgt = wgt16.reshape(N_TOKENS, 16)[:, :TOP_K]
    return idx, wgt


def kernel(x, gate_weight, expert_bias):
    batch, seq, dim = x.shape
    n = batch * seq
    x_flat = x.reshape(n, dim)
    w_t = gate_weight.T
    bias = expert_bias.reshape(1, NUM_EXPERTS)
    return _hybrid_sc(x_flat, w_t, bias)


# speedup vs baseline: 1.4840x; 1.4840x over previous
import functools

import jax
import jax.numpy as jnp
from jax import lax
from jax.experimental import pallas as pl
from jax.experimental.pallas import tpu as pltpu
from jax.experimental.pallas import tpu_sc as plsc

TOP_K = 8
NUM_EXPERTS = 64
BLK = 2048
N_TOKENS = 16384


def _router_body(x_ref, w_ref, b_ref, idx_ref, wgt_ref):
    x = x_ref[...]
    w = w_ref[...]
    logits = jax.lax.dot_general(
        x, w, (((1,), (0,)), ((), ())),
        preferred_element_type=jnp.float32,
    ) + b_ref[...]

    lanes = jax.lax.broadcasted_iota(jnp.int32, logits.shape, 1)
    neg_inf = jnp.float32(-jnp.inf)
    cur = logits
    vals, idxs = [], []
    for _ in range(TOP_K):
        m = jnp.max(cur, axis=1, keepdims=True)
        idx = jnp.argmax(cur, axis=1).reshape(-1, 1)
        vals.append(m)
        idxs.append(idx)
        cur = jnp.where(lanes == idx, neg_inf, cur)

    v = jnp.concatenate(vals, axis=1)
    e = jnp.exp(v - v[:, :1])
    wgt_ref[...] = e / jnp.sum(e, axis=1, keepdims=True)
    idx_ref[...] = jnp.concatenate(idxs, axis=1)


def _fused_tc(x_flat, w_t, bias):
    n, dim = x_flat.shape
    grid = (n // BLK,)
    return pl.pallas_call(
        _router_body,
        grid=grid,
        in_specs=[
            pl.BlockSpec((BLK, dim), lambda i: (i, 0)),
            pl.BlockSpec((dim, NUM_EXPERTS), lambda i: (0, 0)),
            pl.BlockSpec((1, NUM_EXPERTS), lambda i: (0, 0)),
        ],
        out_specs=[
            pl.BlockSpec((BLK, TOP_K), lambda i: (i, 0)),
            pl.BlockSpec((BLK, TOP_K), lambda i: (i, 0)),
        ],
        out_shape=[
            jax.ShapeDtypeStruct((n, TOP_K), jnp.int32),
            jax.ShapeDtypeStruct((n, TOP_K), jnp.float32),
        ],
        compiler_params=pltpu.CompilerParams(
            dimension_semantics=("arbitrary",),
        ),
    )(x_flat, w_t, bias)


def _matmul_body(x_ref, w_ref, b_ref, out_ref):
    out_ref[...] = jax.lax.dot_general(
        x_ref[...], w_ref[...], (((1,), (0,)), ((), ())),
        preferred_element_type=jnp.float32,
    ) + b_ref[...]


def _tc_logits(x_flat, w_t, bias):
    n, dim = x_flat.shape
    return pl.pallas_call(
        _matmul_body,
        grid=(n // BLK,),
        in_specs=[
            pl.BlockSpec((BLK, dim), lambda i: (i, 0)),
            pl.BlockSpec((dim, NUM_EXPERTS), lambda i: (0, 0)),
            pl.BlockSpec((1, NUM_EXPERTS), lambda i: (0, 0)),
        ],
        out_specs=pl.BlockSpec((BLK, NUM_EXPERTS), lambda i: (i, 0)),
        out_shape=jax.ShapeDtypeStruct((n, NUM_EXPERTS), jnp.float32),
        compiler_params=pltpu.CompilerParams(
            dimension_semantics=("arbitrary",),
        ),
    )(x_flat, w_t, bias)


_SCALE = 2097152.0
_NW = 32
_ROWS_W = N_TOKENS // _NW


def _sc_topk_call(logits_flat):
    mesh = plsc.VectorSubcoreMesh(core_axis_name="c", subcore_axis_name="s")
    rows = _ROWS_W

    @functools.partial(
        pl.kernel,
        mesh=mesh,
        out_type=[
            jax.ShapeDtypeStruct((N_TOKENS * 16,), jnp.int32),
            jax.ShapeDtypeStruct((N_TOKENS * 16,), jnp.float32),
        ],
        scratch_types=[
            pltpu.VMEM((rows * 64,), jnp.float32),
            pltpu.VMEM((rows * 16,), jnp.int32),
            pltpu.VMEM((rows * 16,), jnp.float32),
        ],
        compiler_params=pltpu.CompilerParams(needs_layout_passes=False),
    )
    def sc_topk(logits_hbm, idx_hbm, wgt_hbm, log_v, idx_v, wgt_v):
        wid = lax.axis_index("s") * 2 + lax.axis_index("c")
        base = wid * rows
        pltpu.sync_copy(logits_hbm.at[pl.ds(base * 64, rows * 64)], log_v)

        lane = lax.iota(jnp.int32, 16)
        mask8 = lane < 8

        def one_row(r, _):
            chunks = []
            for c in range(4):
                v = log_v[pl.ds(r * 64 + c * 16, 16)]
                v = jnp.minimum(jnp.maximum(v, -15.9), 15.9)
                q = (v * _SCALE).astype(jnp.int32)
                key = q * 64 + (63 - (lane + c * 16))
                skey, _sv = plsc.sort_key_val(key, key, descending=True)
                chunks.append(skey)
            t01 = jnp.maximum(chunks[0], lax.rev(chunks[1], (0,)))
            t23 = jnp.maximum(chunks[2], lax.rev(chunks[3], (0,)))
            t01, _sv = plsc.sort_key_val(t01, t01, descending=True)
            t23, _sv = plsc.sort_key_val(t23, t23, descending=True)
            f = jnp.maximum(t01, lax.rev(t23, (0,)))
            f, _sv = plsc.sort_key_val(f, f, descending=True)

            idx = 63 - (f & 63)
            vv = (f >> 6).astype(jnp.float32) * (1.0 / _SCALE)
            e = jnp.exp(vv - jnp.max(vv))
            e = jnp.where(mask8, e, 0.0)
            w = e / jnp.sum(e)
            idx_v[pl.ds(r * 16, 16)] = idx
            wgt_v[pl.ds(r * 16, 16)] = w
            return _

        lax.fori_loop(0, rows, one_row, 0)
        pltpu.sync_copy(idx_v, idx_hbm.at[pl.ds(base * 16, rows * 16)])
        pltpu.sync_copy(wgt_v, wgt_hbm.at[pl.ds(base * 16, rows * 16)])

    return sc_topk(logits_flat)


def _hybrid_sc(x_flat, w_t, bias):
    logits = _tc_logits(x_flat, w_t, bias)
    idx16, wgt16 = _sc_topk_call(logits.reshape(-1))
    idx = idx16.reshape(N_TOKENS, 16)[:, :TOP_K]
    wgt = wgt16.reshape(N_TOKENS, 16)[:, :TOP_K]
    return idx, wgt


def _router_body2(xa_ref, xb_ref, w_ref, b_ref, idx_ref, wgt_ref):
    w = w_ref[...]
    b = b_ref[...]
    blk = xa_ref.shape[0]
    for half, x_ref in enumerate((xa_ref, xb_ref)):
        logits = jax.lax.dot_general(
            x_ref[...], w, (((1,), (0,)), ((), ())),
            preferred_element_type=jnp.float32,
        ) + b
        lanes = jax.lax.broadcasted_iota(jnp.int32, logits.shape, 1)
        neg_inf = jnp.float32(-jnp.inf)
        cur = logits
        vals, idxs = [], []
        for _ in range(TOP_K):
            m = jnp.max(cur, axis=1, keepdims=True)
            idx = jnp.argmax(cur, axis=1).reshape(-1, 1)
            vals.append(m)
            idxs.append(idx)
            cur = jnp.where(lanes == idx, neg_inf, cur)
        v = jnp.concatenate(vals, axis=1)
        e = jnp.exp(v - v[:, :1])
        sl = pl.ds(half * blk, blk)
        wgt_ref[sl, :] = e / jnp.sum(e, axis=1, keepdims=True)
        idx_ref[sl, :] = jnp.concatenate(idxs, axis=1)


def _fused_tc2(x_flat, w_t, bias, blk=1024):
    n, dim = x_flat.shape
    grid = (n // (2 * blk),)
    return pl.pallas_call(
        _router_body2,
        grid=grid,
        in_specs=[
            pl.BlockSpec((blk, dim), lambda i: (2 * i, 0)),
            pl.BlockSpec((blk, dim), lambda i: (2 * i + 1, 0)),
            pl.BlockSpec((dim, NUM_EXPERTS), lambda i: (0, 0)),
            pl.BlockSpec((1, NUM_EXPERTS), lambda i: (0, 0)),
        ],
        out_specs=[
            pl.BlockSpec((2 * blk, TOP_K), lambda i: (i, 0)),
            pl.BlockSpec((2 * blk, TOP_K), lambda i: (i, 0)),
        ],
        out_shape=[
            jax.ShapeDtypeStruct((n, TOP_K), jnp.int32),
            jax.ShapeDtypeStruct((n, TOP_K), jnp.float32),
        ],
        compiler_params=pltpu.CompilerParams(
            dimension_semantics=("arbitrary",),
        ),
    )(x_flat, x_flat, w_t, bias)


def kernel(x, gate_weight, expert_bias):
    batch, seq, dim = x.shape
    n = batch * seq
    x_flat = x.reshape(n, dim)
    w_t = gate_weight.T
    bias = expert_bias.reshape(1, NUM_EXPERTS)
    return _fused_tc2(x_flat, w_t, bias)


# speedup vs baseline: 1.7565x; 1.1836x over previous
import jax
import jax.numpy as jnp
from jax.experimental import pallas as pl
from jax.experimental.pallas import tpu as pltpu

TOP_K = 8
NUM_EXPERTS = 64
BLK = 2048


def _router_body(x_ref, w_ref, b_ref, idx_ref, wgt_ref):
    x = x_ref[...]
    w = w_ref[...]
    logits = jax.lax.dot_general(
        x, w, (((1,), (0,)), ((), ())),
        preferred_element_type=jnp.float32,
    ) + b_ref[...]

    lanes = jax.lax.broadcasted_iota(jnp.int32, logits.shape, 1)
    neg_inf = jnp.float32(-jnp.inf)
    cur = logits
    vals, idxs = [], []
    for _ in range(TOP_K):
        m = jnp.max(cur, axis=1, keepdims=True)
        idx = jnp.argmax(cur, axis=1).reshape(-1, 1)
        vals.append(m)
        idxs.append(idx)
        cur = jnp.where(lanes == idx, neg_inf, cur)

    v = jnp.concatenate(vals, axis=1)
    e = jnp.exp(v - v[:, :1])
    wgt_ref[...] = e / jnp.sum(e, axis=1, keepdims=True)
    idx_ref[...] = jnp.concatenate(idxs, axis=1)


def kernel(x, gate_weight, expert_bias):
    batch, seq, dim = x.shape
    n = batch * seq
    x_flat = x.reshape(n, dim)
    w_t = gate_weight.T
    bias = expert_bias.reshape(1, NUM_EXPERTS)

    return pl.pallas_call(
        _router_body,
        grid=(n // BLK,),
        in_specs=[
            pl.BlockSpec((BLK, dim), lambda i: (i, 0)),
            pl.BlockSpec((dim, NUM_EXPERTS), lambda i: (0, 0)),
            pl.BlockSpec((1, NUM_EXPERTS), lambda i: (0, 0)),
        ],
        out_specs=[
            pl.BlockSpec((BLK, TOP_K), lambda i: (i, 0)),
            pl.BlockSpec((BLK, TOP_K), lambda i: (i, 0)),
        ],
        out_shape=[
            jax.ShapeDtypeStruct((n, TOP_K), jnp.int32),
            jax.ShapeDtypeStruct((n, TOP_K), jnp.float32),
        ],
        compiler_params=pltpu.CompilerParams(
            dimension_semantics=("arbitrary",),
        ),
    )(x_flat, w_t, bias)
